# baseline (speedup 1.0000x reference)
import functools

import jax
import jax.numpy as jnp
from jax import lax
from jax.experimental import pallas as pl
from jax.experimental.pallas import tpu as pltpu
from jax.experimental.pallas import tpu_sc as plsc

N_NODES = 10000
NPAD = 10240
N_EDGES = 320000
CHUNK = 128
EROWS = N_EDGES // CHUNK
NWORKERS = 32
ROWS_MAIN = EROWS // NWORKERS
FULL = ROWS_MAIN * NWORKERS
HID = 64

_mesh = plsc.VectorSubcoreMesh(core_axis_name="c", subcore_axis_name="s")
_sc_params = pltpu.CompilerParams(use_tc_tiling_on_sc=False,
                                  needs_layout_passes=False)


def _rsqrt16(x):
    i = lax.bitcast_convert_type(x, jnp.int32)
    i = jnp.int32(0x5F3759DF) - lax.shift_right_arithmetic(i, 1)
    y = lax.bitcast_convert_type(i, jnp.float32)
    for _ in range(3):
        y = y * (1.5 - 0.5 * x * y * y)
    return y


TPC = EROWS // 16
FULL1 = TPC * 16
DROWS = NPAD // NWORKERS


@functools.partial(
    pl.kernel,
    out_type=jax.ShapeDtypeStruct((NPAD, HID), jnp.float32),
    mesh=_mesh,
    compiler_params=_sc_params,
    scratch_types=[
        pltpu.VMEM((TPC, CHUNK), jnp.int32),
        pltpu.VMEM((1, CHUNK), jnp.int32),
        pltpu.VMEM((NPAD,), jnp.float32),
        pltpu.VMEM((16, DROWS), jnp.float32),
        pltpu.VMEM((DROWS,), jnp.float32),
        pltpu.VMEM((DROWS, HID), jnp.float32),
        pltpu.VMEM_SHARED((16, NPAD), jnp.float32),
        pltpu.SemaphoreType.DMA,
    ],
)
def _dinv_kernel(edge_hbm, dv_out, dstbuf, xdst, degbuf, sbuf, degv, dvbuf,
                 sacc, ssem):
    cid = lax.axis_index("c")
    sid = lax.axis_index("s")

    def zfill(i, _):
        degbuf[pl.ds(i * 16, 16)] = jnp.zeros((16,), jnp.float32)
        return 0

    lax.fori_loop(0, NPAD // 16, zfill, 0)
    pltpu.sync_copy(edge_hbm.at[1, pl.ds(sid * TPC, TPC)], dstbuf)

    ones16 = jnp.ones((16,), jnp.float32)

    def chunk(c, _):
        for k in range(CHUNK // 16):
            idx = dstbuf[c, pl.ds(k * 16, 16)]
            plsc.addupdate_scatter(degbuf, [idx], ones16)
        return 0

    lax.fori_loop(0, TPC, chunk, 0)

    @pl.when(sid < EROWS - FULL1)
    def _():
        pltpu.sync_copy(edge_hbm.at[1, pl.ds(FULL1 + sid, 1)], xdst)
        for k in range(CHUNK // 16):
            idx = xdst[0, pl.ds(k * 16, 16)]
            plsc.addupdate_scatter(degbuf, [idx], ones16)

    pltpu.sync_copy(degbuf, sacc.at[sid])
    plsc.subcore_barrier()

    row0 = cid * (NPAD // 2) + sid * DROWS
    for t in range(16):
        pltpu.async_copy(sacc.at[t, pl.ds(row0, DROWS)], sbuf.at[t], ssem)
    for t in range(16):
        pltpu.make_async_copy(sacc.at[t, pl.ds(row0, DROWS)], sbuf.at[t],
                              ssem).wait()

    def dcomp(i, _):
        v = jnp.ones((16,), jnp.float32)
        for t in range(16):
            v = v + sbuf[t, pl.ds(i * 16, 16)]
        degv[pl.ds(i * 16, 16)] = _rsqrt16(v)
        return 0

    lax.fori_loop(0, DROWS // 16, dcomp, 0)

    def brow(r, _):
        v = plsc.load_gather(degv, [jnp.full((16,), r, jnp.int32)])
        for k in range(HID // 16):
            dvbuf[r, pl.ds(k * 16, 16)] = v
        return 0

    lax.fori_loop(0, DROWS, brow, 0)
    pltpu.sync_copy(dvbuf, dv_out.at[pl.ds(row0, DROWS)])


@functools.partial(
    pl.kernel,
    out_type=jax.ShapeDtypeStruct((2, NPAD, HID), jnp.float32),
    mesh=_mesh,
    compiler_params=_sc_params,
    scratch_types=[
        pltpu.VMEM((ROWS_MAIN, CHUNK), jnp.int32),
        pltpu.VMEM((ROWS_MAIN, CHUNK), jnp.int32),
        pltpu.VMEM((1, CHUNK), jnp.int32),
        pltpu.VMEM((1, CHUNK), jnp.int32),
        [pltpu.VMEM((CHUNK, HID), jnp.float32)] * 2,
        [pltpu.SemaphoreType.DMA] * 2,
        [pltpu.SemaphoreType.DMA] * 2,
        pltpu.VMEM_SHARED((NPAD, HID), jnp.float32),
        pltpu.VMEM_SHARED((N_NODES, HID), jnp.float32),
    ],
)
def _scatter_kernel(g_hbm, edge_hbm, out_hbm,
                    srcbuf, dstbuf, xsrc, xdst, rows, gsem, ssem, acc, gsp):
    cid = lax.axis_index("c")
    sid = lax.axis_index("s")
    wid = sid * 2 + cid

    def zrow(r, _):
        def zcol(k, _):
            rows[0][r, pl.ds(k * 16, 16)] = jnp.zeros((16,), jnp.float32)
            return 0
        return lax.fori_loop(0, HID // 16, zcol, 0)

    lax.fori_loop(0, CHUNK, zrow, 0)
    seg = NPAD // 16
    for k in range(seg // CHUNK):
        pltpu.sync_copy(rows[0], acc.at[pl.ds(sid * seg + k * CHUNK, CHUNK)])

    pltpu.sync_copy(edge_hbm.at[0, pl.ds(wid * ROWS_MAIN, ROWS_MAIN)], srcbuf)
    pltpu.sync_copy(edge_hbm.at[1, pl.ds(wid * ROWS_MAIN, ROWS_MAIN)], dstbuf)
    nseg = N_NODES // 16
    pltpu.sync_copy(g_hbm.at[pl.ds(sid * nseg, nseg)], gsp.at[pl.ds(sid * nseg, nseg)])
    plsc.subcore_barrier()

    def fire_g(c, b):
        pltpu.async_copy(gsp.at[srcbuf.at[c]], rows[b], gsem[b])

    def wait_g(c, b):
        pltpu.make_async_copy(gsp.at[srcbuf.at[c]], rows[b], gsem[b]).wait()

    def fire_s(c, b):
        pltpu.async_copy(rows[b], acc.at[dstbuf.at[c]], ssem[b], add=True)

    def wait_s(c, b):
        pltpu.make_async_copy(rows[b], acc.at[dstbuf.at[c]], ssem[b]).wait()

    fire_g(0, 0)

    def body(i, _):
        for j in range(2):
            c = 2 * i + j
            b = j
            wait_g(c, b)
            fire_s(c, b)

            @pl.when(c >= 1)
            def _():
                wait_s(c - 1, 1 - b)

            @pl.when(c + 1 < ROWS_MAIN)
            def _():
                fire_g(c + 1, 1 - b)
        return 0

    lax.fori_loop(0, ROWS_MAIN // 2, body, 0)
    wait_s(ROWS_MAIN - 1, (ROWS_MAIN - 1) % 2)

    @pl.when(wid < EROWS - FULL)
    def _():
        pltpu.sync_copy(edge_hbm.at[0, pl.ds(FULL + wid, 1)], xsrc)
        pltpu.sync_copy(edge_hbm.at[1, pl.ds(FULL + wid, 1)], xdst)
        pltpu.async_copy(gsp.at[xsrc.at[0]], rows[0], gsem[0])
        pltpu.make_async_copy(gsp.at[xsrc.at[0]], rows[0], gsem[0]).wait()
        pltpu.sync_copy(rows[0], acc.at[xdst.at[0]], add=True)

    plsc.subcore_barrier()
    pltpu.sync_copy(acc.at[pl.ds(sid * seg, seg)],
                    out_hbm.at[cid, pl.ds(sid * seg, seg)])


RP = 1000
NP2 = N_NODES // 2


def _mm_body(x_ref, w_ref, o_ref):
    o_ref[...] = jnp.dot(x_ref[...], w_ref[...],
                         preferred_element_type=jnp.float32)


def _scale_body(dv_ref, h_ref, o_ref):
    o_ref[...] = dv_ref[...] * h_ref[...]


def _tc2_body(dv_ref, s_ref, g_ref, b_ref, o_ref):
    dv = dv_ref[...]
    s = s_ref[...]
    z = jnp.maximum(dv * (s[0] + s[1] + g_ref[...]) + b_ref[...], 0.0)
    o_ref[...] = dv * z


def _tc3_body(dv_ref, s_ref, g_ref, w_ref, b_ref, o_ref):
    s = s_ref[...]
    t = dv_ref[...] * (s[0] + s[1] + g_ref[...])
    v = jnp.dot(t, w_ref[...],
                preferred_element_type=jnp.float32) + b_ref[...]
    o_ref[...] = v.reshape(2 * RP, 128)


_dv_spec = pl.BlockSpec((RP, 128), lambda i: (i, 0))
_s_spec = pl.BlockSpec((2, RP, 128), lambda i: (0, i, 0))
_g_spec = pl.BlockSpec((RP, 128), lambda i: (i, 0))

_mm = pl.pallas_call(
    _mm_body,
    grid=(NP2 // RP,),
    in_specs=[pl.BlockSpec((RP, 256), lambda i: (i, 0)),
              pl.BlockSpec((256, 128), lambda i: (0, 0))],
    out_specs=pl.BlockSpec((RP, 128), lambda i: (i, 0)),
    out_shape=jax.ShapeDtypeStruct((NP2, 128), jnp.float32),
)

_scale = pl.pallas_call(
    _scale_body,
    grid=(NP2 // RP,),
    in_specs=[_dv_spec, _g_spec],
    out_specs=pl.BlockSpec((RP, 128), lambda i: (i, 0)),
    out_shape=jax.ShapeDtypeStruct((NP2, 128), jnp.float32),
)

_tc2 = pl.pallas_call(
    _tc2_body,
    grid=(NP2 // RP,),
    in_specs=[_dv_spec,
              _s_spec,
              _g_spec,
              pl.BlockSpec((1, 128), lambda i: (0, 0))],
    out_specs=pl.BlockSpec((RP, 128), lambda i: (i, 0)),
    out_shape=jax.ShapeDtypeStruct((NP2, 128), jnp.float32),
)

_tc3 = pl.pallas_call(
    _tc3_body,
    grid=(NP2 // RP,),
    in_specs=[_dv_spec,
              _s_spec,
              _g_spec,
              pl.BlockSpec((128, 256), lambda i: (0, 0)),
              pl.BlockSpec((1, 256), lambda i: (0, 0))],
    out_specs=pl.BlockSpec((2 * RP, 128), lambda i: (i, 0)),
    out_shape=jax.ShapeDtypeStruct((N_NODES, 128), jnp.float32),
)


def _blockdiag2(w):
    r, c = w.shape
    z = jnp.zeros((r, c), w.dtype)
    return jnp.concatenate(
        [jnp.concatenate([w, z], axis=1),
         jnp.concatenate([z, w], axis=1)], axis=0)


def kernel(x, edge_index, W1, b1, W2, b2):
    edge_r = edge_index.astype(jnp.int32).reshape(2, EROWS, CHUNK)

    dvu = _dinv_kernel(edge_r)
    dvp = dvu.reshape(NPAD // 2, 128)

    xp = x.reshape(NP2, 256)
    w1d = _blockdiag2(W1)
    h1p = _mm(xp, w1d)
    g1p = _scale(dvp, h1p)

    s1 = _scatter_kernel(g1p.reshape(N_NODES, HID), edge_r)
    s1p = s1.reshape(2, NPAD // 2, 128)

    b1p = jnp.tile(b1, 2).reshape(1, 128)
    g2p = _tc2(dvp, s1p, g1p, b1p)

    s2 = _scatter_kernel(g2p.reshape(N_NODES, HID), edge_r)
    s2p = s2.reshape(2, NPAD // 2, 128)

    w2d = _blockdiag2(W2)
    b2p = jnp.tile(b2, 2).reshape(1, 256)
    return _tc3(dvp, s2p, g2p, w2d, b2p)


# speedup vs baseline: 1.0229x; 1.0229x over previous
import functools

import jax
import jax.numpy as jnp
from jax import lax
from jax.experimental import pallas as pl
from jax.experimental.pallas import tpu as pltpu
from jax.experimental.pallas import tpu_sc as plsc

N_NODES = 10000
NPAD = 10240
N_EDGES = 320000
CHUNK = 128
EROWS = N_EDGES // CHUNK
NWORKERS = 32
ROWS_MAIN = EROWS // NWORKERS
FULL = ROWS_MAIN * NWORKERS
HID = 64

_mesh = plsc.VectorSubcoreMesh(core_axis_name="c", subcore_axis_name="s")
_sc_params = pltpu.CompilerParams(use_tc_tiling_on_sc=False,
                                  needs_layout_passes=False)


def _rsqrt16(x):
    i = lax.bitcast_convert_type(x, jnp.int32)
    i = jnp.int32(0x5F3759DF) - lax.shift_right_arithmetic(i, 1)
    y = lax.bitcast_convert_type(i, jnp.float32)
    for _ in range(3):
        y = y * (1.5 - 0.5 * x * y * y)
    return y


TPC = EROWS // 16
FULL1 = TPC * 16
DROWS = NPAD // NWORKERS


@functools.partial(
    pl.kernel,
    out_type=jax.ShapeDtypeStruct((NPAD, HID), jnp.float32),
    mesh=_mesh,
    compiler_params=_sc_params,
    scratch_types=[
        pltpu.VMEM((TPC, CHUNK), jnp.int32),
        pltpu.VMEM((1, CHUNK), jnp.int32),
        pltpu.VMEM((NPAD // 16,), jnp.float32),
        pltpu.VMEM((CHUNK,), jnp.float32),
        pltpu.VMEM((DROWS,), jnp.float32),
        pltpu.VMEM((DROWS, HID), jnp.float32),
        pltpu.VMEM_SHARED((NPAD,), jnp.float32),
        pltpu.SemaphoreType.DMA,
    ],
)
def _dinv_kernel(edge_hbm, dv_out, dstbuf, xdst, zbuf, ones, degv, dvbuf, dacc,
                 hsem):
    cid = lax.axis_index("c")
    sid = lax.axis_index("s")

    def zfill(i, _):
        zbuf[pl.ds(i * 16, 16)] = jnp.zeros((16,), jnp.float32)
        return 0

    lax.fori_loop(0, (NPAD // 16) // 16, zfill, 0)

    def ofill(i, _):
        ones[pl.ds(i * 16, 16)] = jnp.ones((16,), jnp.float32)
        return 0

    lax.fori_loop(0, CHUNK // 16, ofill, 0)

    seg = NPAD // 16
    pltpu.sync_copy(zbuf, dacc.at[pl.ds(sid * seg, seg)])
    pltpu.sync_copy(edge_hbm.at[1, pl.ds(sid * TPC, TPC)], dstbuf)
    plsc.subcore_barrier()

    def chunk(c, _):
        pltpu.async_copy(ones, dacc.at[dstbuf.at[c]], hsem, add=True)
        return 0

    lax.fori_loop(0, TPC, chunk, 0)

    @pl.when(sid < EROWS - FULL1)
    def _():
        pltpu.sync_copy(edge_hbm.at[1, pl.ds(FULL1 + sid, 1)], xdst)
        pltpu.sync_copy(ones, dacc.at[xdst.at[0]], add=True)

    def hdrain(c, _):
        pltpu.make_async_copy(ones, dacc.at[dstbuf.at[c]], hsem).wait()
        return 0

    lax.fori_loop(0, TPC, hdrain, 0)
    plsc.subcore_barrier()

    row0 = cid * (NPAD // 2) + sid * DROWS
    pltpu.sync_copy(dacc.at[pl.ds(row0, DROWS)], degv)

    def dcomp(i, _):
        v = degv[pl.ds(i * 16, 16)] + 1.0
        degv[pl.ds(i * 16, 16)] = _rsqrt16(v)
        return 0

    lax.fori_loop(0, DROWS // 16, dcomp, 0)

    def brow(r, _):
        v = plsc.load_gather(degv, [jnp.full((16,), r, jnp.int32)])
        for k in range(HID // 16):
            dvbuf[r, pl.ds(k * 16, 16)] = v
        return 0

    lax.fori_loop(0, DROWS, brow, 0)
    pltpu.sync_copy(dvbuf, dv_out.at[pl.ds(row0, DROWS)])


@functools.partial(
    pl.kernel,
    out_type=jax.ShapeDtypeStruct((2, NPAD, HID), jnp.float32),
    mesh=_mesh,
    compiler_params=_sc_params,
    scratch_types=[
        pltpu.VMEM((ROWS_MAIN, CHUNK), jnp.int32),
        pltpu.VMEM((ROWS_MAIN, CHUNK), jnp.int32),
        pltpu.VMEM((1, CHUNK), jnp.int32),
        pltpu.VMEM((1, CHUNK), jnp.int32),
        [pltpu.VMEM((CHUNK, HID), jnp.float32)] * 2,
        [pltpu.SemaphoreType.DMA] * 2,
        [pltpu.SemaphoreType.DMA] * 2,
        pltpu.VMEM_SHARED((NPAD, HID), jnp.float32),
        pltpu.VMEM_SHARED((N_NODES, HID), jnp.float32),
    ],
)
def _scatter_kernel(g_hbm, edge_hbm, out_hbm,
                    srcbuf, dstbuf, xsrc, xdst, rows, gsem, ssem, acc, gsp):
    cid = lax.axis_index("c")
    sid = lax.axis_index("s")
    wid = sid * 2 + cid

    def zrow(r, _):
        def zcol(k, _):
            rows[0][r, pl.ds(k * 16, 16)] = jnp.zeros((16,), jnp.float32)
            return 0
        return lax.fori_loop(0, HID // 16, zcol, 0)

    lax.fori_loop(0, CHUNK, zrow, 0)
    seg = NPAD // 16
    for k in range(seg // CHUNK):
        pltpu.sync_copy(rows[0], acc.at[pl.ds(sid * seg + k * CHUNK, CHUNK)])

    pltpu.sync_copy(edge_hbm.at[0, pl.ds(wid * ROWS_MAIN, ROWS_MAIN)], srcbuf)
    pltpu.sync_copy(edge_hbm.at[1, pl.ds(wid * ROWS_MAIN, ROWS_MAIN)], dstbuf)
    nseg = N_NODES // 16
    pltpu.sync_copy(g_hbm.at[pl.ds(sid * nseg, nseg)], gsp.at[pl.ds(sid * nseg, nseg)])
    plsc.subcore_barrier()

    def fire_g(c, b):
        pltpu.async_copy(gsp.at[srcbuf.at[c]], rows[b], gsem[b])

    def wait_g(c, b):
        pltpu.make_async_copy(gsp.at[srcbuf.at[c]], rows[b], gsem[b]).wait()

    def fire_s(c, b):
        pltpu.async_copy(rows[b], acc.at[dstbuf.at[c]], ssem[b], add=True)

    def wait_s(c, b):
        pltpu.make_async_copy(rows[b], acc.at[dstbuf.at[c]], ssem[b]).wait()

    fire_g(0, 0)

    def body(i, _):
        for j in range(2):
            c = 2 * i + j
            b = j
            wait_g(c, b)
            fire_s(c, b)

            @pl.when(c >= 1)
            def _():
                wait_s(c - 1, 1 - b)

            @pl.when(c + 1 < ROWS_MAIN)
            def _():
                fire_g(c + 1, 1 - b)
        return 0

    lax.fori_loop(0, ROWS_MAIN // 2, body, 0)
    wait_s(ROWS_MAIN - 1, (ROWS_MAIN - 1) % 2)

    @pl.when(wid < EROWS - FULL)
    def _():
        pltpu.sync_copy(edge_hbm.at[0, pl.ds(FULL + wid, 1)], xsrc)
        pltpu.sync_copy(edge_hbm.at[1, pl.ds(FULL + wid, 1)], xdst)
        pltpu.async_copy(gsp.at[xsrc.at[0]], rows[0], gsem[0])
        pltpu.make_async_copy(gsp.at[xsrc.at[0]], rows[0], gsem[0]).wait()
        pltpu.sync_copy(rows[0], acc.at[xdst.at[0]], add=True)

    plsc.subcore_barrier()
    pltpu.sync_copy(acc.at[pl.ds(sid * seg, seg)],
                    out_hbm.at[cid, pl.ds(sid * seg, seg)])


RP = 1000
NP2 = N_NODES // 2


def _mm_body(x_ref, w_ref, o_ref):
    o_ref[...] = jnp.dot(x_ref[...], w_ref[...],
                         preferred_element_type=jnp.float32)


def _scale_body(dv_ref, h_ref, o_ref):
    o_ref[...] = dv_ref[...] * h_ref[...]


def _tc2_body(dv_ref, s_ref, g_ref, b_ref, o_ref):
    dv = dv_ref[...]
    s = s_ref[...]
    z = jnp.maximum(dv * (s[0] + s[1] + g_ref[...]) + b_ref[...], 0.0)
    o_ref[...] = dv * z


def _tc3_body(dv_ref, s_ref, g_ref, w_ref, b_ref, o_ref):
    s = s_ref[...]
    t = dv_ref[...] * (s[0] + s[1] + g_ref[...])
    v = jnp.dot(t, w_ref[...],
                preferred_element_type=jnp.float32) + b_ref[...]
    o_ref[...] = v.reshape(2 * RP, 128)


_dv_spec = pl.BlockSpec((RP, 128), lambda i: (i, 0))
_s_spec = pl.BlockSpec((2, RP, 128), lambda i: (0, i, 0))
_g_spec = pl.BlockSpec((RP, 128), lambda i: (i, 0))

_mm = pl.pallas_call(
    _mm_body,
    grid=(NP2 // RP,),
    in_specs=[pl.BlockSpec((RP, 256), lambda i: (i, 0)),
              pl.BlockSpec((256, 128), lambda i: (0, 0))],
    out_specs=pl.BlockSpec((RP, 128), lambda i: (i, 0)),
    out_shape=jax.ShapeDtypeStruct((NP2, 128), jnp.float32),
)

_scale = pl.pallas_call(
    _scale_body,
    grid=(NP2 // RP,),
    in_specs=[_dv_spec, _g_spec],
    out_specs=pl.BlockSpec((RP, 128), lambda i: (i, 0)),
    out_shape=jax.ShapeDtypeStruct((NP2, 128), jnp.float32),
)

_tc2 = pl.pallas_call(
    _tc2_body,
    grid=(NP2 // RP,),
    in_specs=[_dv_spec,
              _s_spec,
              _g_spec,
              pl.BlockSpec((1, 128), lambda i: (0, 0))],
    out_specs=pl.BlockSpec((RP, 128), lambda i: (i, 0)),
    out_shape=jax.ShapeDtypeStruct((NP2, 128), jnp.float32),
)

_tc3 = pl.pallas_call(
    _tc3_body,
    grid=(NP2 // RP,),
    in_specs=[_dv_spec,
              _s_spec,
              _g_spec,
              pl.BlockSpec((128, 256), lambda i: (0, 0)),
              pl.BlockSpec((1, 256), lambda i: (0, 0))],
    out_specs=pl.BlockSpec((2 * RP, 128), lambda i: (i, 0)),
    out_shape=jax.ShapeDtypeStruct((N_NODES, 128), jnp.float32),
)


def _blockdiag2(w):
    r, c = w.shape
    z = jnp.zeros((r, c), w.dtype)
    return jnp.concatenate(
        [jnp.concatenate([w, z], axis=1),
         jnp.concatenate([z, w], axis=1)], axis=0)


def kernel(x, edge_index, W1, b1, W2, b2):
    edge_r = edge_index.astype(jnp.int32).reshape(2, EROWS, CHUNK)

    dvu = _dinv_kernel(edge_r)
    dvp = dvu.reshape(NPAD // 2, 128)

    xp = x.reshape(NP2, 256)
    w1d = _blockdiag2(W1)
    h1p = _mm(xp, w1d)
    g1p = _scale(dvp, h1p)

    s1 = _scatter_kernel(g1p.reshape(N_NODES, HID), edge_r)
    s1p = s1.reshape(2, NPAD // 2, 128)

    b1p = jnp.tile(b1, 2).reshape(1, 128)
    g2p = _tc2(dvp, s1p, g1p, b1p)

    s2 = _scatter_kernel(g2p.reshape(N_NODES, HID), edge_r)
    s2p = s2.reshape(2, NPAD // 2, 128)

    w2d = _blockdiag2(W2)
    b2p = jnp.tile(b2, 2).reshape(1, 256)
    return _tc3(dvp, s2p, g2p, w2d, b2p)


# speedup vs baseline: 1.1457x; 1.1201x over previous
import functools

import jax
import jax.numpy as jnp
from jax import lax
from jax.experimental import pallas as pl
from jax.experimental.pallas import tpu as pltpu
from jax.experimental.pallas import tpu_sc as plsc

N_NODES = 10000
NPAD = 10240
N_EDGES = 320000
CHUNK = 128
EROWS = N_EDGES // CHUNK
NWORKERS = 32
ROWS_MAIN = EROWS // NWORKERS
FULL = ROWS_MAIN * NWORKERS
HID = 64

_mesh = plsc.VectorSubcoreMesh(core_axis_name="c", subcore_axis_name="s")
_sc_params = pltpu.CompilerParams(use_tc_tiling_on_sc=False,
                                  needs_layout_passes=False)


def _rsqrt16(x):
    i = lax.bitcast_convert_type(x, jnp.int32)
    i = jnp.int32(0x5F3759DF) - lax.shift_right_arithmetic(i, 1)
    y = lax.bitcast_convert_type(i, jnp.float32)
    for _ in range(3):
        y = y * (1.5 - 0.5 * x * y * y)
    return y


TPC = EROWS // 16
FULL1 = TPC * 16
DROWS = NPAD // NWORKERS


@functools.partial(
    pl.kernel,
    out_type=jax.ShapeDtypeStruct((NPAD, HID), jnp.float32),
    mesh=_mesh,
    compiler_params=_sc_params,
    scratch_types=[
        pltpu.VMEM((TPC, CHUNK), jnp.int32),
        pltpu.VMEM((1, CHUNK), jnp.int32),
        pltpu.VMEM((NPAD // 16,), jnp.float32),
        pltpu.VMEM((CHUNK,), jnp.float32),
        pltpu.VMEM((DROWS,), jnp.float32),
        pltpu.VMEM((DROWS, HID), jnp.float32),
        pltpu.VMEM_SHARED((NPAD,), jnp.float32),
        pltpu.SemaphoreType.DMA,
    ],
)
def _dinv_kernel(edge_hbm, dv_out, dstbuf, xdst, zbuf, ones, degv, dvbuf, dacc,
                 hsem):
    cid = lax.axis_index("c")
    sid = lax.axis_index("s")

    def zfill(i, _):
        zbuf[pl.ds(i * 16, 16)] = jnp.zeros((16,), jnp.float32)
        return 0

    lax.fori_loop(0, (NPAD // 16) // 16, zfill, 0)

    def ofill(i, _):
        ones[pl.ds(i * 16, 16)] = jnp.ones((16,), jnp.float32)
        return 0

    lax.fori_loop(0, CHUNK // 16, ofill, 0)

    seg = NPAD // 16
    pltpu.sync_copy(zbuf, dacc.at[pl.ds(sid * seg, seg)])
    pltpu.sync_copy(edge_hbm.at[1, pl.ds(sid * TPC, TPC)], dstbuf)
    plsc.subcore_barrier()

    def chunk(c, _):
        pltpu.async_copy(ones, dacc.at[dstbuf.at[c]], hsem, add=True)
        return 0

    lax.fori_loop(0, TPC, chunk, 0)

    @pl.when(sid < EROWS - FULL1)
    def _():
        pltpu.sync_copy(edge_hbm.at[1, pl.ds(FULL1 + sid, 1)], xdst)
        pltpu.sync_copy(ones, dacc.at[xdst.at[0]], add=True)

    def hdrain(c, _):
        pltpu.make_async_copy(ones, dacc.at[dstbuf.at[c]], hsem).wait()
        return 0

    lax.fori_loop(0, TPC, hdrain, 0)
    plsc.subcore_barrier()

    row0 = cid * (NPAD // 2) + sid * DROWS
    pltpu.sync_copy(dacc.at[pl.ds(row0, DROWS)], degv)

    def dcomp(i, _):
        v = degv[pl.ds(i * 16, 16)] + 1.0
        degv[pl.ds(i * 16, 16)] = _rsqrt16(v)
        return 0

    lax.fori_loop(0, DROWS // 16, dcomp, 0)

    def brow(r, _):
        v = plsc.load_gather(degv, [jnp.full((16,), r, jnp.int32)])
        for k in range(HID // 16):
            dvbuf[r, pl.ds(k * 16, 16)] = v
        return 0

    lax.fori_loop(0, DROWS, brow, 0)
    pltpu.sync_copy(dvbuf, dv_out.at[pl.ds(row0, DROWS)])


@functools.partial(
    pl.kernel,
    out_type=jax.ShapeDtypeStruct((2, NPAD, HID), jnp.float32),
    mesh=_mesh,
    compiler_params=_sc_params,
    scratch_types=[
        pltpu.VMEM((ROWS_MAIN, CHUNK), jnp.int32),
        pltpu.VMEM((ROWS_MAIN, CHUNK), jnp.int32),
        pltpu.VMEM((1, CHUNK), jnp.int32),
        pltpu.VMEM((1, CHUNK), jnp.int32),
        [pltpu.VMEM((CHUNK, HID), jnp.float32)] * 3,
        [pltpu.SemaphoreType.DMA] * 3,
        [pltpu.SemaphoreType.DMA] * 3,
        pltpu.VMEM_SHARED((NPAD, HID), jnp.float32),
        pltpu.VMEM_SHARED((N_NODES, HID), jnp.float32),
    ],
)
def _scatter_kernel(g_hbm, edge_hbm, out_hbm,
                    srcbuf, dstbuf, xsrc, xdst, rows, gsem, ssem, acc, gsp):
    cid = lax.axis_index("c")
    sid = lax.axis_index("s")
    wid = sid * 2 + cid

    def zrow(r, _):
        def zcol(k, _):
            rows[0][r, pl.ds(k * 16, 16)] = jnp.zeros((16,), jnp.float32)
            return 0
        return lax.fori_loop(0, HID // 16, zcol, 0)

    lax.fori_loop(0, CHUNK, zrow, 0)
    seg = NPAD // 16
    for k in range(seg // CHUNK):
        pltpu.sync_copy(rows[0], acc.at[pl.ds(sid * seg + k * CHUNK, CHUNK)])

    pltpu.sync_copy(edge_hbm.at[0, pl.ds(wid * ROWS_MAIN, ROWS_MAIN)], srcbuf)
    pltpu.sync_copy(edge_hbm.at[1, pl.ds(wid * ROWS_MAIN, ROWS_MAIN)], dstbuf)
    nseg = N_NODES // 16
    pltpu.sync_copy(g_hbm.at[pl.ds(sid * nseg, nseg)], gsp.at[pl.ds(sid * nseg, nseg)])
    plsc.subcore_barrier()

    def fire_g(c, b):
        pltpu.async_copy(gsp.at[srcbuf.at[c]], rows[b], gsem[b])

    def wait_g(c, b):
        pltpu.make_async_copy(gsp.at[srcbuf.at[c]], rows[b], gsem[b]).wait()

    def fire_s(c, b):
        pltpu.async_copy(rows[b], acc.at[dstbuf.at[c]], ssem[b], add=True)

    def wait_s(c, b):
        pltpu.make_async_copy(rows[b], acc.at[dstbuf.at[c]], ssem[b]).wait()

    fire_g(0, 0)
    fire_g(1, 1)

    def body(i, _):
        for j in range(3):
            c = 3 * i + j
            b = j
            bn = (j + 2) % 3
            wait_g(c, b)
            fire_s(c, b)

            @pl.when(c >= 1)
            def _():
                wait_s(c - 1, bn)

            @pl.when(c + 2 < ROWS_MAIN)
            def _():
                fire_g(c + 2, bn)
        return 0

    lax.fori_loop(0, ROWS_MAIN // 3, body, 0)
    wait_s(ROWS_MAIN - 1, (ROWS_MAIN - 1) % 3)

    @pl.when(wid < EROWS - FULL)
    def _():
        pltpu.sync_copy(edge_hbm.at[0, pl.ds(FULL + wid, 1)], xsrc)
        pltpu.sync_copy(edge_hbm.at[1, pl.ds(FULL + wid, 1)], xdst)
        pltpu.async_copy(gsp.at[xsrc.at[0]], rows[0], gsem[0])
        pltpu.make_async_copy(gsp.at[xsrc.at[0]], rows[0], gsem[0]).wait()
        pltpu.sync_copy(rows[0], acc.at[xdst.at[0]], add=True)

    plsc.subcore_barrier()
    pltpu.sync_copy(acc.at[pl.ds(sid * seg, seg)],
                    out_hbm.at[cid, pl.ds(sid * seg, seg)])


RP = 1000
NP2 = N_NODES // 2


def _mm_body(x_ref, w_ref, o_ref):
    o_ref[...] = jnp.dot(x_ref[...], w_ref[...],
                         preferred_element_type=jnp.float32)


def _scale_body(dv_ref, h_ref, o_ref):
    o_ref[...] = dv_ref[...] * h_ref[...]


def _tc2_body(dv_ref, s_ref, g_ref, b_ref, o_ref):
    dv = dv_ref[...]
    s = s_ref[...]
    z = jnp.maximum(dv * (s[0] + s[1] + g_ref[...]) + b_ref[...], 0.0)
    o_ref[...] = dv * z


def _tc3_body(dv_ref, s_ref, g_ref, w_ref, b_ref, o_ref):
    s = s_ref[...]
    t = dv_ref[...] * (s[0] + s[1] + g_ref[...])
    v = jnp.dot(t, w_ref[...],
                preferred_element_type=jnp.float32) + b_ref[...]
    o_ref[...] = v.reshape(2 * RP, 128)


_dv_spec = pl.BlockSpec((RP, 128), lambda i: (i, 0))
_s_spec = pl.BlockSpec((2, RP, 128), lambda i: (0, i, 0))
_g_spec = pl.BlockSpec((RP, 128), lambda i: (i, 0))

_mm = pl.pallas_call(
    _mm_body,
    grid=(NP2 // RP,),
    in_specs=[pl.BlockSpec((RP, 256), lambda i: (i, 0)),
              pl.BlockSpec((256, 128), lambda i: (0, 0))],
    out_specs=pl.BlockSpec((RP, 128), lambda i: (i, 0)),
    out_shape=jax.ShapeDtypeStruct((NP2, 128), jnp.float32),
)

_scale = pl.pallas_call(
    _scale_body,
    grid=(NP2 // RP,),
    in_specs=[_dv_spec, _g_spec],
    out_specs=pl.BlockSpec((RP, 128), lambda i: (i, 0)),
    out_shape=jax.ShapeDtypeStruct((NP2, 128), jnp.float32),
)

_tc2 = pl.pallas_call(
    _tc2_body,
    grid=(NP2 // RP,),
    in_specs=[_dv_spec,
              _s_spec,
              _g_spec,
              pl.BlockSpec((1, 128), lambda i: (0, 0))],
    out_specs=pl.BlockSpec((RP, 128), lambda i: (i, 0)),
    out_shape=jax.ShapeDtypeStruct((NP2, 128), jnp.float32),
)

_tc3 = pl.pallas_call(
    _tc3_body,
    grid=(NP2 // RP,),
    in_specs=[_dv_spec,
              _s_spec,
              _g_spec,
              pl.BlockSpec((128, 256), lambda i: (0, 0)),
              pl.BlockSpec((1, 256), lambda i: (0, 0))],
    out_specs=pl.BlockSpec((2 * RP, 128), lambda i: (i, 0)),
    out_shape=jax.ShapeDtypeStruct((N_NODES, 128), jnp.float32),
)


def _blockdiag2(w):
    r, c = w.shape
    z = jnp.zeros((r, c), w.dtype)
    return jnp.concatenate(
        [jnp.concatenate([w, z], axis=1),
         jnp.concatenate([z, w], axis=1)], axis=0)


def kernel(x, edge_index, W1, b1, W2, b2):
    edge_r = edge_index.astype(jnp.int32).reshape(2, EROWS, CHUNK)

    dvu = _dinv_kernel(edge_r)
    dvp = dvu.reshape(NPAD // 2, 128)

    xp = x.reshape(NP2, 256)
    w1d = _blockdiag2(W1)
    h1p = _mm(xp, w1d)
    g1p = _scale(dvp, h1p)

    s1 = _scatter_kernel(g1p.reshape(N_NODES, HID), edge_r)
    s1p = s1.reshape(2, NPAD // 2, 128)

    b1p = jnp.tile(b1, 2).reshape(1, 128)
    g2p = _tc2(dvp, s1p, g1p, b1p)

    s2 = _scatter_kernel(g2p.reshape(N_NODES, HID), edge_r)
    s2p = s2.reshape(2, NPAD // 2, 128)

    w2d = _blockdiag2(W2)
    b2p = jnp.tile(b2, 2).reshape(1, 256)
    return _tc3(dvp, s2p, g2p, w2d, b2p)
